# baseline (device time: 1229118 ns/iter reference)
import jax
import jax.numpy as jnp
from jax import lax
from jax.experimental import pallas as pl
from jax.experimental.pallas import tpu as pltpu

NC = 16


def _rsdw_kernel(x, dy):
    M, D = x.shape
    _, F = dy.shape
    FH = F // 2
    CF = FH // NC
    DH = D // 2

    def body(x_ref, dy_ref, out_ref, dy_buf, p_buf, yrecv, rout,
             dy_sems, out_sems, ysend_sems, yrecv_sems,
             xsend_sems, xrecv_sems):
        my_x = lax.axis_index("x")
        my_y = lax.axis_index("y")
        y_nbr = (my_x, 1 - my_y)
        x_nbr = (1 - my_x, my_y)

        mine = my_y * DH
        theirs = (1 - my_y) * DH
        fbase = my_x * FH

        def dydesc(c):
            return pltpu.make_async_copy(
                dy_ref.at[:, pl.ds(fbase + c * CF, CF)],
                dy_buf.at[c % 2], dy_sems.at[c % 2],
            )

        def ydesc(c):
            return pltpu.make_async_remote_copy(
                src_ref=p_buf.at[c % 2, pl.ds(theirs, DH), :],
                dst_ref=yrecv.at[c],
                send_sem=ysend_sems.at[c],
                recv_sem=yrecv_sems.at[c],
                device_id=y_nbr,
                device_id_type=pl.DeviceIdType.MESH,
            )

        def xdesc(c):
            return pltpu.make_async_remote_copy(
                src_ref=rout.at[c % 2],
                dst_ref=out_ref.at[:, pl.ds(fbase + c * CF, CF)],
                send_sem=xsend_sems.at[c],
                recv_sem=xrecv_sems.at[c],
                device_id=x_nbr,
                device_id_type=pl.DeviceIdType.MESH,
            )

        def odesc(c):
            return pltpu.make_async_copy(
                rout.at[c % 2],
                out_ref.at[:, pl.ds(fbase + c * CF, CF)],
                out_sems.at[c % 2],
            )

        def reduce_and_send(c):
            rout[c % 2] = p_buf[c % 2, pl.ds(mine, DH), :] + yrecv[c]
            xdesc(c).start()
            odesc(c).start()

        dydesc(0).start()

        barrier_sem = pltpu.get_barrier_semaphore()
        for nbr in (y_nbr, x_nbr):
            pl.semaphore_signal(
                barrier_sem, inc=1,
                device_id=nbr, device_id_type=pl.DeviceIdType.MESH,
            )
        pl.semaphore_wait(barrier_sem, 2)

        def step(c, _):
            dydesc(c).wait()
            pl.when(c >= 2)(lambda: ydesc(c - 2).wait_send())
            p_buf[c % 2] = lax.dot_general(
                x_ref[...], dy_buf[c % 2],
                dimension_numbers=(((0,), (0,)), ((), ())),
                preferred_element_type=jnp.float32,
            )
            ydesc(c).start()
            pl.when(c < NC - 1)(lambda: dydesc(c + 1).start())

            def b_stage():
                def guard():
                    odesc(c - 3).wait()
                    xdesc(c - 3).wait_send()
                pl.when(c >= 3)(guard)
                ydesc(c - 1).wait_recv()
                reduce_and_send(c - 1)
            pl.when(c >= 1)(b_stage)
            return 0

        lax.fori_loop(0, NC, step, 0)

        k = NC - 1
        odesc(k - 2).wait()
        xdesc(k - 2).wait_send()
        ydesc(k).wait_recv()
        reduce_and_send(k)
        for c in (NC - 2, NC - 1):
            ydesc(c).wait_send()
            xdesc(c).wait_send()
            odesc(c).wait()
        lax.fori_loop(0, NC, lambda c, _: (xdesc(c).wait_recv(), 0)[1], 0)

    out_shape = jax.ShapeDtypeStruct((DH, F), jnp.float32)
    return pl.pallas_call(
        body,
        out_shape=out_shape,
        in_specs=[
            pl.BlockSpec(memory_space=pltpu.MemorySpace.VMEM),
            pl.BlockSpec(memory_space=pltpu.MemorySpace.HBM),
        ],
        out_specs=pl.BlockSpec(memory_space=pltpu.MemorySpace.HBM),
        scratch_shapes=[
            pltpu.VMEM((2, M, CF), jnp.float32),
            pltpu.VMEM((2, D, CF), jnp.float32),
            pltpu.VMEM((NC, DH, CF), jnp.float32),
            pltpu.VMEM((2, DH, CF), jnp.float32),
            pltpu.SemaphoreType.DMA((2,)),
            pltpu.SemaphoreType.DMA((2,)),
            pltpu.SemaphoreType.DMA((NC,)),
            pltpu.SemaphoreType.DMA((NC,)),
            pltpu.SemaphoreType.DMA((NC,)),
            pltpu.SemaphoreType.DMA((NC,)),
        ],
        compiler_params=pltpu.CompilerParams(collective_id=0),
    )(x, dy)


def kernel(x, dy):
    out = _rsdw_kernel(x, dy)
    return _hbm_copy(out)


NSPLIT = 8


def _hbm_copy(t):
    rows = t.shape[0] // NSPLIT

    def body(in_ref, out_ref, sems):
        for i in range(NSPLIT):
            pltpu.make_async_copy(
                in_ref.at[pl.ds(i * rows, rows), :],
                out_ref.at[pl.ds(i * rows, rows), :],
                sems.at[i],
            ).start()
        for i in range(NSPLIT):
            pltpu.make_async_copy(
                in_ref.at[pl.ds(i * rows, rows), :],
                out_ref.at[pl.ds(i * rows, rows), :],
                sems.at[i],
            ).wait()

    return pl.pallas_call(
        body,
        out_shape=jax.ShapeDtypeStruct(t.shape, t.dtype),
        in_specs=[pl.BlockSpec(memory_space=pltpu.MemorySpace.HBM)],
        out_specs=pl.BlockSpec(memory_space=pltpu.MemorySpace.HBM),
        scratch_shapes=[pltpu.SemaphoreType.DMA((NSPLIT,))],
    )(t)


# device time: 229337 ns/iter; 5.3594x vs baseline; 5.3594x over previous
import jax
import jax.numpy as jnp
from jax import lax
from jax.experimental import pallas as pl
from jax.experimental.pallas import tpu as pltpu

NC = 16


def _rsdw_kernel(x, dy):
    M, D = x.shape
    _, F = dy.shape
    FH = F // 2
    CF = FH // NC
    DH = D // 2

    def body(x_ref, dy_ref, out_ref, dy_buf, p_buf, yrecv, rout,
             dy_sems, out_sems, ysend_sems, yrecv_sems,
             xsend_sems, xrecv_sems):
        my_x = lax.axis_index("x")
        my_y = lax.axis_index("y")
        y_nbr = (my_x, 1 - my_y)
        x_nbr = (1 - my_x, my_y)

        mine = my_y * DH
        theirs = (1 - my_y) * DH
        fbase = my_x * FH

        def dydesc(c):
            return pltpu.make_async_copy(
                dy_ref.at[:, pl.ds(fbase + c * CF, CF)],
                dy_buf.at[c % 2], dy_sems.at[c % 2],
            )

        def ydesc(c):
            return pltpu.make_async_remote_copy(
                src_ref=p_buf.at[c % 2, pl.ds(theirs, DH), :],
                dst_ref=yrecv.at[c],
                send_sem=ysend_sems.at[c],
                recv_sem=yrecv_sems.at[c],
                device_id=y_nbr,
                device_id_type=pl.DeviceIdType.MESH,
            )

        def xdesc(c):
            return pltpu.make_async_remote_copy(
                src_ref=rout.at[c % 2],
                dst_ref=out_ref.at[:, pl.ds(fbase + c * CF, CF)],
                send_sem=xsend_sems.at[c],
                recv_sem=xrecv_sems.at[c],
                device_id=x_nbr,
                device_id_type=pl.DeviceIdType.MESH,
            )

        def odesc(c):
            return pltpu.make_async_copy(
                rout.at[c % 2],
                out_ref.at[:, pl.ds(fbase + c * CF, CF)],
                out_sems.at[c % 2],
            )

        def reduce_and_send(c):
            rout[c % 2] = p_buf[c % 2, pl.ds(mine, DH), :] + yrecv[c]
            xdesc(c).start()
            odesc(c).start()

        dydesc(0).start()

        barrier_sem = pltpu.get_barrier_semaphore()
        for nbr in (y_nbr, x_nbr):
            pl.semaphore_signal(
                barrier_sem, inc=1,
                device_id=nbr, device_id_type=pl.DeviceIdType.MESH,
            )
        pl.semaphore_wait(barrier_sem, 2)

        def step(c, _):
            dydesc(c).wait()
            pl.when(c >= 2)(lambda: ydesc(c - 2).wait_send())
            p_buf[c % 2] = lax.dot_general(
                x_ref[...], dy_buf[c % 2],
                dimension_numbers=(((0,), (0,)), ((), ())),
                preferred_element_type=jnp.float32,
            )
            ydesc(c).start()
            pl.when(c < NC - 1)(lambda: dydesc(c + 1).start())

            def b_stage():
                def guard():
                    odesc(c - 3).wait()
                    xdesc(c - 3).wait_send()
                pl.when(c >= 3)(guard)
                ydesc(c - 1).wait_recv()
                reduce_and_send(c - 1)
            pl.when(c >= 1)(b_stage)
            return 0

        lax.fori_loop(0, NC, step, 0)

        k = NC - 1
        odesc(k - 2).wait()
        xdesc(k - 2).wait_send()
        ydesc(k).wait_recv()
        reduce_and_send(k)
        for c in (NC - 2, NC - 1):
            ydesc(c).wait_send()
            xdesc(c).wait_send()
            odesc(c).wait()
        lax.fori_loop(0, NC, lambda c, _: (xdesc(c).wait_recv(), 0)[1], 0)

    out_shape = jax.ShapeDtypeStruct((DH, F), jnp.float32)
    return pl.pallas_call(
        body,
        out_shape=out_shape,
        in_specs=[
            pl.BlockSpec(memory_space=pltpu.MemorySpace.VMEM),
            pl.BlockSpec(memory_space=pltpu.MemorySpace.HBM),
        ],
        out_specs=pl.BlockSpec(memory_space=pltpu.MemorySpace.HBM),
        scratch_shapes=[
            pltpu.VMEM((2, M, CF), jnp.float32),
            pltpu.VMEM((2, D, CF), jnp.float32),
            pltpu.VMEM((NC, DH, CF), jnp.float32),
            pltpu.VMEM((2, DH, CF), jnp.float32),
            pltpu.SemaphoreType.DMA((2,)),
            pltpu.SemaphoreType.DMA((2,)),
            pltpu.SemaphoreType.DMA((NC,)),
            pltpu.SemaphoreType.DMA((NC,)),
            pltpu.SemaphoreType.DMA((NC,)),
            pltpu.SemaphoreType.DMA((NC,)),
        ],
        compiler_params=pltpu.CompilerParams(collective_id=0),
    )(x, dy)


def kernel(x, dy):
    out = _rsdw_kernel(x, dy)
    return _hbm_copy(out)


NB = 8
NSLOT = 4


def _hbm_copy(t):
    rows = t.shape[0] // NB
    cols = t.shape[1]

    def body(in_ref, out_ref, buf, in_sems, out_sems):
        def indesc(i):
            return pltpu.make_async_copy(
                in_ref.at[pl.ds(i * rows, rows), :],
                buf.at[i % NSLOT], in_sems.at[i % NSLOT],
            )

        def outdesc(i):
            return pltpu.make_async_copy(
                buf.at[i % NSLOT],
                out_ref.at[pl.ds(i * rows, rows), :],
                out_sems.at[i % NSLOT],
            )

        for j in range(min(NSLOT, NB)):
            indesc(j).start()
        for i in range(NB):
            indesc(i).wait()
            outdesc(i).start()
            if i + NSLOT < NB:
                outdesc(i).wait()
                indesc(i + NSLOT).start()
        for i in range(max(NB - NSLOT, 0), NB):
            outdesc(i).wait()

    return pl.pallas_call(
        body,
        out_shape=jax.ShapeDtypeStruct(t.shape, t.dtype),
        in_specs=[pl.BlockSpec(memory_space=pltpu.MemorySpace.HBM)],
        out_specs=pl.BlockSpec(memory_space=pltpu.MemorySpace.HBM),
        scratch_shapes=[
            pltpu.VMEM((NSLOT, rows, cols), jnp.float32),
            pltpu.SemaphoreType.DMA((NSLOT,)),
            pltpu.SemaphoreType.DMA((NSLOT,)),
        ],
    )(t)


# device time: 229270 ns/iter; 5.3610x vs baseline; 1.0003x over previous
import jax
import jax.numpy as jnp
from jax import lax
from jax.experimental import pallas as pl
from jax.experimental.pallas import tpu as pltpu

NC = 16


def _rsdw_kernel(x, dy):
    M, D = x.shape
    _, F = dy.shape
    FH = F // 2
    CF = FH // NC
    DH = D // 2

    def body(x_ref, dy_ref, out_ref, x_vmem, dy_buf, p_buf, yrecv, rout,
             x_sem, dy_sems, out_sems, ysend_sems, yrecv_sems,
             xsend_sems, xrecv_sems):
        my_x = lax.axis_index("x")
        my_y = lax.axis_index("y")
        y_nbr = (my_x, 1 - my_y)
        x_nbr = (1 - my_x, my_y)

        mine = my_y * DH
        theirs = (1 - my_y) * DH
        fbase = my_x * FH

        def dydesc(c):
            return pltpu.make_async_copy(
                dy_ref.at[:, pl.ds(fbase + c * CF, CF)],
                dy_buf.at[c % 2], dy_sems.at[c % 2],
            )

        def ydesc(c):
            return pltpu.make_async_remote_copy(
                src_ref=p_buf.at[c % 2, pl.ds(theirs, DH), :],
                dst_ref=yrecv.at[c],
                send_sem=ysend_sems.at[c],
                recv_sem=yrecv_sems.at[c],
                device_id=y_nbr,
                device_id_type=pl.DeviceIdType.MESH,
            )

        def xdesc(c):
            return pltpu.make_async_remote_copy(
                src_ref=rout.at[c % 2],
                dst_ref=out_ref.at[:, pl.ds(fbase + c * CF, CF)],
                send_sem=xsend_sems.at[c],
                recv_sem=xrecv_sems.at[c],
                device_id=x_nbr,
                device_id_type=pl.DeviceIdType.MESH,
            )

        def odesc(c):
            return pltpu.make_async_copy(
                rout.at[c % 2],
                out_ref.at[:, pl.ds(fbase + c * CF, CF)],
                out_sems.at[c % 2],
            )

        def reduce_and_send(c):
            rout[c % 2] = p_buf[c % 2, pl.ds(mine, DH), :] + yrecv[c]
            xdesc(c).start()
            odesc(c).start()

        xload = pltpu.make_async_copy(x_ref, x_vmem, x_sem)
        xload.start()
        dydesc(0).start()

        barrier_sem = pltpu.get_barrier_semaphore()
        for nbr in (y_nbr, x_nbr):
            pl.semaphore_signal(
                barrier_sem, inc=1,
                device_id=nbr, device_id_type=pl.DeviceIdType.MESH,
            )
        pl.semaphore_wait(barrier_sem, 2)

        def step(c, _):
            dydesc(c).wait()
            pl.when(c >= 2)(lambda: ydesc(c - 2).wait_send())
            pl.when(c == 0)(
                lambda: pltpu.make_async_copy(x_ref, x_vmem, x_sem).wait())
            p_buf[c % 2] = lax.dot_general(
                x_vmem[...], dy_buf[c % 2],
                dimension_numbers=(((0,), (0,)), ((), ())),
                preferred_element_type=jnp.float32,
            )
            ydesc(c).start()
            pl.when(c < NC - 1)(lambda: dydesc(c + 1).start())

            def b_stage():
                def guard():
                    odesc(c - 3).wait()
                    xdesc(c - 3).wait_send()
                pl.when(c >= 3)(guard)
                ydesc(c - 1).wait_recv()
                reduce_and_send(c - 1)
            pl.when(c >= 1)(b_stage)
            return 0

        lax.fori_loop(0, NC, step, 0)

        k = NC - 1
        odesc(k - 2).wait()
        xdesc(k - 2).wait_send()
        ydesc(k).wait_recv()
        reduce_and_send(k)
        for c in (NC - 2, NC - 1):
            ydesc(c).wait_send()
            xdesc(c).wait_send()
            odesc(c).wait()
        lax.fori_loop(0, NC, lambda c, _: (xdesc(c).wait_recv(), 0)[1], 0)

    out_shape = jax.ShapeDtypeStruct((DH, F), jnp.float32)
    return pl.pallas_call(
        body,
        out_shape=out_shape,
        in_specs=[
            pl.BlockSpec(memory_space=pltpu.MemorySpace.HBM),
            pl.BlockSpec(memory_space=pltpu.MemorySpace.HBM),
        ],
        out_specs=pl.BlockSpec(memory_space=pltpu.MemorySpace.HBM),
        scratch_shapes=[
            pltpu.VMEM((M, D), jnp.float32),
            pltpu.VMEM((2, M, CF), jnp.float32),
            pltpu.VMEM((2, D, CF), jnp.float32),
            pltpu.VMEM((NC, DH, CF), jnp.float32),
            pltpu.VMEM((2, DH, CF), jnp.float32),
            pltpu.SemaphoreType.DMA,
            pltpu.SemaphoreType.DMA((2,)),
            pltpu.SemaphoreType.DMA((2,)),
            pltpu.SemaphoreType.DMA((NC,)),
            pltpu.SemaphoreType.DMA((NC,)),
            pltpu.SemaphoreType.DMA((NC,)),
            pltpu.SemaphoreType.DMA((NC,)),
        ],
        compiler_params=pltpu.CompilerParams(
            collective_id=0,
            vmem_limit_bytes=60 * 1024 * 1024,
        ),
    )(x, dy)


def kernel(x, dy):
    out = _rsdw_kernel(x, dy)
    return _hbm_copy(out)


NB = 8
NSLOT = 8


def _hbm_copy(t):
    rows = t.shape[0] // NB
    cols = t.shape[1]

    def body(in_ref, out_ref, buf, in_sems, out_sems):
        def indesc(i):
            return pltpu.make_async_copy(
                in_ref.at[pl.ds(i * rows, rows), :],
                buf.at[i % NSLOT], in_sems.at[i % NSLOT],
            )

        def outdesc(i):
            return pltpu.make_async_copy(
                buf.at[i % NSLOT],
                out_ref.at[pl.ds(i * rows, rows), :],
                out_sems.at[i % NSLOT],
            )

        for j in range(min(NSLOT, NB)):
            indesc(j).start()
        for i in range(NB):
            indesc(i).wait()
            outdesc(i).start()
            if i + NSLOT < NB:
                outdesc(i).wait()
                indesc(i + NSLOT).start()
        for i in range(max(NB - NSLOT, 0), NB):
            outdesc(i).wait()

    return pl.pallas_call(
        body,
        out_shape=jax.ShapeDtypeStruct(t.shape, t.dtype),
        in_specs=[pl.BlockSpec(memory_space=pltpu.MemorySpace.HBM)],
        out_specs=pl.BlockSpec(memory_space=pltpu.MemorySpace.HBM),
        scratch_shapes=[
            pltpu.VMEM((NSLOT, rows, cols), jnp.float32),
            pltpu.SemaphoreType.DMA((NSLOT,)),
            pltpu.SemaphoreType.DMA((NSLOT,)),
        ],
        compiler_params=pltpu.CompilerParams(
            vmem_limit_bytes=48 * 1024 * 1024,
        ),
    )(t)
